# baseline (speedup 1.0000x reference)
import functools
import jax
import jax.numpy as jnp
from jax import lax
from jax.experimental import pallas as pl
from jax.experimental.pallas import tpu as pltpu
from jax.experimental.pallas import tpu_sc as plsc

N = 16384
B = 16
TILE = 2048
GRID = N // TILE
Z128 = 128
LANES = 16
NSUB = 16
CHUNK = N // NSUB


def _tc_body(z_ref, pos_ref, emb_ref, Wp_ref, W1_ref, b1_ref, W2_ref,
             b2_ref, xa_ref, A_sc, P_sc):
    i = pl.program_id(0)

    @pl.when(i == 0)
    def _init():
        A_sc[...] = jnp.dot(emb_ref[...], W1_ref[...],
                            preferred_element_type=jnp.float32)
        P_sc[...] = jnp.dot(Wp_ref[...], W1_ref[...],
                            preferred_element_type=jnp.float32)

    zc = z_ref[0]
    lane = lax.broadcasted_iota(jnp.int32, (TILE, Z128), 1)
    oh_z = (zc == lane).astype(jnp.float32)
    a = jnp.dot(oh_z, A_sc[...], preferred_element_type=jnp.float32)
    p = jnp.dot(pos_ref[...], P_sc[...], preferred_element_type=jnp.float32)
    hpre = a + p + b1_ref[...]
    h = hpre * jax.nn.sigmoid(hpre)
    xa_ref[...] = (jnp.dot(h, W2_ref[...], preferred_element_type=jnp.float32)
                   + b2_ref[...])


def _tc_energies(z, pos, emb, Wp, W1, b1, W2, b2):
    D = emb.shape[1]
    H = W1.shape[1]
    emb_p = jnp.pad(emb, ((0, Z128 - emb.shape[0]), (0, 0)))
    pos_p = jnp.pad(pos, ((0, 0), (0, 5)))
    Wp_p = jnp.pad(Wp, ((0, 5), (0, 0)))
    z_in = z.reshape(GRID, TILE, 1).astype(jnp.int32)
    b1r = b1.reshape(1, H)
    b2r = b2.reshape(1, 1)

    return pl.pallas_call(
        _tc_body,
        grid=(GRID,),
        in_specs=[
            pl.BlockSpec((1, TILE, 1), lambda i: (i, 0, 0)),
            pl.BlockSpec((TILE, 8), lambda i: (i, 0)),
            pl.BlockSpec((Z128, D), lambda i: (0, 0)),
            pl.BlockSpec((8, D), lambda i: (0, 0)),
            pl.BlockSpec((D, H), lambda i: (0, 0)),
            pl.BlockSpec((1, H), lambda i: (0, 0)),
            pl.BlockSpec((H, 1), lambda i: (0, 0)),
            pl.BlockSpec((1, 1), lambda i: (0, 0)),
        ],
        out_specs=pl.BlockSpec((TILE, 1), lambda i: (i, 0)),
        out_shape=jax.ShapeDtypeStruct((N, 1), jnp.float32),
        scratch_shapes=[
            pltpu.VMEM((Z128, H), jnp.float32),
            pltpu.VMEM((8, H), jnp.float32),
        ],
    )(z_in, pos_p, emb_p, Wp_p, W1, b1r, W2, b2r)


_MESH = plsc.VectorSubcoreMesh(core_axis_name="c", subcore_axis_name="s")


@functools.partial(
    pl.kernel,
    out_type=jax.ShapeDtypeStruct((LANES,), jnp.float32),
    mesh=_MESH,
    compiler_params=pltpu.CompilerParams(needs_layout_passes=False),
    scratch_types=[
        pltpu.VMEM((CHUNK,), jnp.float32),
        pltpu.VMEM((CHUNK,), jnp.int32),
        pltpu.VMEM((B, LANES), jnp.float32),
        pltpu.VMEM((NSUB, B, LANES), jnp.float32),
        pltpu.VMEM((LANES,), jnp.float32),
        pltpu.VMEM_SHARED((NSUB, B, LANES), jnp.float32),
    ],
)
def _sc_segsum(xa_hbm, ids_hbm, out_hbm, xa_v, ids_v, part_v, gather_v,
               res_v, shared):
    cid = lax.axis_index("c")
    sid = lax.axis_index("s")

    @pl.when(cid == 0)
    def _work():
        base = sid * CHUNK
        pltpu.sync_copy(xa_hbm.at[pl.ds(base, CHUNK)], xa_v)
        pltpu.sync_copy(ids_hbm.at[pl.ds(base, CHUNK)], ids_v)

        def body(i, accs):
            v = xa_v[pl.ds(i * LANES, LANES)]
            d = ids_v[pl.ds(i * LANES, LANES)]
            return tuple(accs[b] + jnp.where(d == b, v, 0.0)
                         for b in range(B))

        init = tuple(jnp.zeros((LANES,), jnp.float32) for _ in range(B))
        accs = lax.fori_loop(0, CHUNK // LANES, body, init)

        for b in range(B):
            part_v[b] = accs[b]
        pltpu.sync_copy(part_v, shared.at[sid])

    plsc.subcore_barrier()

    @pl.when((cid == 0) & (sid == 0))
    def _out():
        pltpu.sync_copy(shared, gather_v)
        for b in range(B):
            m = gather_v[0, b]
            for t in range(1, NSUB):
                m = m + gather_v[t, b]
            part_v[b] = m
        row = lax.iota(jnp.int32, LANES)
        total = jnp.zeros((LANES,), jnp.float32)
        for l in range(LANES):
            col = jnp.full((LANES,), l, jnp.int32)
            total = total + plsc.load_gather(part_v, [row, col])
        res_v[...] = total
        pltpu.sync_copy(res_v, out_hbm)


def kernel(z, pos, batch, emb, Wp, W1, b1, W2, b2):
    xa = _tc_energies(z, pos, emb, Wp, W1, b1, W2, b2)
    out = _sc_segsum(xa.reshape(N), batch.astype(jnp.int32))
    return out.reshape(B, 1)


# speedup vs baseline: 1.1835x; 1.1835x over previous
import functools
import jax
import jax.numpy as jnp
from jax import lax
from jax.experimental import pallas as pl
from jax.experimental.pallas import tpu as pltpu
from jax.experimental.pallas import tpu_sc as plsc

N = 16384
B = 16
TILE = 2048
GRID = N // TILE
Z128 = 128
LANES = 16
NSUB = 16
CHUNK = N // NSUB


def _tc_body(z_ref, pos_ref, emb_ref, Wp_ref, W1_ref, b1_ref, W2_ref,
             b2_ref, xa_ref, A_sc, P_sc):
    i = pl.program_id(0)

    @pl.when(i == 0)
    def _init():
        A_sc[...] = jnp.dot(emb_ref[...], W1_ref[...],
                            preferred_element_type=jnp.float32)
        P_sc[...] = jnp.dot(Wp_ref[...], W1_ref[...],
                            preferred_element_type=jnp.float32)

    zc = z_ref[0]
    lane = lax.broadcasted_iota(jnp.int32, (TILE, Z128), 1)
    oh_z = (zc == lane).astype(jnp.float32)
    a = jnp.dot(oh_z, A_sc[...], preferred_element_type=jnp.float32)
    p = jnp.dot(pos_ref[...], P_sc[...], preferred_element_type=jnp.float32)
    hpre = a + p + b1_ref[...]
    h = hpre * jax.nn.sigmoid(hpre)
    xa_row = jax.lax.dot_general(
        W2_ref[...], h, (((0,), (1,)), ((), ())),
        preferred_element_type=jnp.float32)
    xa_ref[0] = xa_row + b2_ref[...]


def _tc_energies(z, pos, emb, Wp, W1, b1, W2, b2):
    D = emb.shape[1]
    H = W1.shape[1]
    emb_p = jnp.pad(emb, ((0, Z128 - emb.shape[0]), (0, 0)))
    pos_p = jnp.pad(pos, ((0, 0), (0, 5)))
    Wp_p = jnp.pad(Wp, ((0, 5), (0, 0)))
    z_in = z.reshape(GRID, TILE, 1).astype(jnp.int32)
    b1r = b1.reshape(1, H)
    b2r = b2.reshape(1, 1)

    return pl.pallas_call(
        _tc_body,
        grid=(GRID,),
        in_specs=[
            pl.BlockSpec((1, TILE, 1), lambda i: (i, 0, 0)),
            pl.BlockSpec((TILE, 8), lambda i: (i, 0)),
            pl.BlockSpec((Z128, D), lambda i: (0, 0)),
            pl.BlockSpec((8, D), lambda i: (0, 0)),
            pl.BlockSpec((D, H), lambda i: (0, 0)),
            pl.BlockSpec((1, H), lambda i: (0, 0)),
            pl.BlockSpec((H, 1), lambda i: (0, 0)),
            pl.BlockSpec((1, 1), lambda i: (0, 0)),
        ],
        out_specs=pl.BlockSpec((1, 1, TILE), lambda i: (i, 0, 0)),
        out_shape=jax.ShapeDtypeStruct((GRID, 1, TILE), jnp.float32),
        scratch_shapes=[
            pltpu.VMEM((Z128, H), jnp.float32),
            pltpu.VMEM((8, H), jnp.float32),
        ],
    )(z_in, pos_p, emb_p, Wp_p, W1, b1r, W2, b2r)


_MESH = plsc.VectorSubcoreMesh(core_axis_name="c", subcore_axis_name="s")


@functools.partial(
    pl.kernel,
    out_type=jax.ShapeDtypeStruct((LANES,), jnp.float32),
    mesh=_MESH,
    compiler_params=pltpu.CompilerParams(needs_layout_passes=False),
    scratch_types=[
        pltpu.VMEM((CHUNK,), jnp.float32),
        pltpu.VMEM((CHUNK,), jnp.int32),
        pltpu.VMEM((B, LANES), jnp.float32),
        pltpu.VMEM((NSUB, B, LANES), jnp.float32),
        pltpu.VMEM((LANES,), jnp.float32),
        pltpu.VMEM_SHARED((NSUB, B, LANES), jnp.float32),
    ],
)
def _sc_segsum(xa_hbm, ids_hbm, out_hbm, xa_v, ids_v, part_v, gather_v,
               res_v, shared):
    cid = lax.axis_index("c")
    sid = lax.axis_index("s")

    @pl.when(cid == 0)
    def _work():
        base = sid * CHUNK
        pltpu.sync_copy(xa_hbm.at[pl.ds(base, CHUNK)], xa_v)
        pltpu.sync_copy(ids_hbm.at[pl.ds(base, CHUNK)], ids_v)

        def body(i, accs):
            v = xa_v[pl.ds(i * LANES, LANES)]
            d = ids_v[pl.ds(i * LANES, LANES)]
            return tuple(accs[b] + jnp.where(d == b, v, 0.0)
                         for b in range(B))

        init = tuple(jnp.zeros((LANES,), jnp.float32) for _ in range(B))
        accs = lax.fori_loop(0, CHUNK // LANES, body, init)

        for b in range(B):
            part_v[b] = accs[b]
        pltpu.sync_copy(part_v, shared.at[sid])

    plsc.subcore_barrier()

    @pl.when((cid == 0) & (sid == 0))
    def _out():
        pltpu.sync_copy(shared, gather_v)
        for b in range(B):
            m = gather_v[0, b]
            for t in range(1, NSUB):
                m = m + gather_v[t, b]
            part_v[b] = m
        row = lax.iota(jnp.int32, LANES)
        total = jnp.zeros((LANES,), jnp.float32)
        for l in range(LANES):
            col = jnp.full((LANES,), l, jnp.int32)
            total = total + plsc.load_gather(part_v, [row, col])
        res_v[...] = total
        pltpu.sync_copy(res_v, out_hbm)


def kernel(z, pos, batch, emb, Wp, W1, b1, W2, b2):
    xa = _tc_energies(z, pos, emb, Wp, W1, b1, W2, b2)
    out = _sc_segsum(xa.reshape(N), batch.astype(jnp.int32))
    return out.reshape(B, 1)
